# baseline (speedup 1.0000x reference)
import functools

import jax
import jax.numpy as jnp
from jax import lax
from jax.experimental import pallas as pl
from jax.experimental.pallas import tpu as pltpu
from jax.experimental.pallas import tpu_sc as plsc

_NC = 2
_NS = 16
_NW = _NC * _NS
_L = 16


def _mm_bias_body(x_ref, w_ref, b_ref, o_ref):
    o_ref[...] = jnp.dot(x_ref[...], w_ref[...],
                         preferred_element_type=jnp.float32) + b_ref[...]


def _mm(x, w, b, block_rows):
    M, K = x.shape
    _, Nc = w.shape
    return pl.pallas_call(
        _mm_bias_body,
        grid=(M // block_rows,),
        in_specs=[pl.BlockSpec((block_rows, K), lambda i: (i, 0)),
                  pl.BlockSpec((K, Nc), lambda i: (0, 0)),
                  pl.BlockSpec((1, Nc), lambda i: (0, 0))],
        out_specs=pl.BlockSpec((block_rows, Nc), lambda i: (i, 0)),
        out_shape=jax.ShapeDtypeStruct((M, Nc), jnp.float32),
    )(x, w, b)


def _mm4_body(x_ref, w_ref, b_ref, o1, o2, o3, o4):
    d = o1.shape[1]
    r = jnp.dot(x_ref[...], w_ref[...],
                preferred_element_type=jnp.float32) + b_ref[...]
    o1[...] = r[:, 0 * d:1 * d]
    o2[...] = r[:, 1 * d:2 * d]
    o3[...] = r[:, 2 * d:3 * d]
    o4[...] = r[:, 3 * d:4 * d]


def _mm4(x, w, b, block_rows):
    M, K = x.shape
    _, Nc = w.shape
    d = Nc // 4
    spec = pl.BlockSpec((block_rows, d), lambda i: (i, 0))
    return pl.pallas_call(
        _mm4_body,
        grid=(M // block_rows,),
        in_specs=[pl.BlockSpec((block_rows, K), lambda i: (i, 0)),
                  pl.BlockSpec((K, Nc), lambda i: (0, 0)),
                  pl.BlockSpec((1, Nc), lambda i: (0, 0))],
        out_specs=[spec, spec, spec, spec],
        out_shape=[jax.ShapeDtypeStruct((M, d), jnp.float32)] * 4,
    )(x, w, b)


def _mm2_body(x_ref, w_ref, b_ref, o1, o2):
    d = o1.shape[1]
    r = jnp.dot(x_ref[...], w_ref[...],
                preferred_element_type=jnp.float32) + b_ref[...]
    o1[...] = r[:, 0 * d:1 * d]
    o2[...] = r[:, 1 * d:2 * d]


def _mm2(x, w, b, block_rows):
    M, K = x.shape
    _, Nc = w.shape
    d = Nc // 2
    spec = pl.BlockSpec((block_rows, d), lambda i: (i, 0))
    return pl.pallas_call(
        _mm2_body,
        grid=(M // block_rows,),
        in_specs=[pl.BlockSpec((block_rows, K), lambda i: (i, 0)),
                  pl.BlockSpec((K, Nc), lambda i: (0, 0)),
                  pl.BlockSpec((1, Nc), lambda i: (0, 0))],
        out_specs=[spec, spec],
        out_shape=[jax.ShapeDtypeStruct((M, d), jnp.float32)] * 2,
    )(x, w, b)


def _final_body(s_ref, hd_ref, acc_ref, w_ref, o_ref):
    prod = hd_ref[...] * acc_ref[...]
    o_ref[...] = s_ref[...] + jnp.dot(prod, w_ref[...],
                                      preferred_element_type=jnp.float32)


def _final(self_out, h_dst, acc, w_neigh_t, block_rows):
    M, D = self_out.shape
    return pl.pallas_call(
        _final_body,
        grid=(M // block_rows,),
        in_specs=[pl.BlockSpec((block_rows, D), lambda i: (i, 0)),
                  pl.BlockSpec((block_rows, D), lambda i: (i, 0)),
                  pl.BlockSpec((block_rows, D), lambda i: (i, 0)),
                  pl.BlockSpec((D, D), lambda i: (0, 0))],
        out_specs=pl.BlockSpec((block_rows, D), lambda i: (i, 0)),
        out_shape=jax.ShapeDtypeStruct((M, D), jnp.float32),
    )(self_out, h_dst, acc, w_neigh_t)


_AC = 128
_ATAIL = 8


def _logits_body(hs_hbm, hd_hbm, ew_hbm, src_hbm, dst_hbm, watt_hbm, alpha_hbm,
                 w_hbm,
                 sidx, didx, s_rows, d_rows, ew_rows, wout,
                 sidx_t, didx_t, s_rows_t, d_rows_t, ew_rows_t, wout_t,
                 watt_v, alpha_v, sem_s, sem_d):
    D = hs_hbm.shape[1]
    E = src_hbm.shape[0]
    epw = E // _NW
    nchunk = (epw - _ATAIL) // _AC
    c = lax.axis_index("c")
    s = lax.axis_index("s")
    wid = s * _NC + c
    base = wid * epw

    pltpu.sync_copy(watt_hbm, watt_v)
    pltpu.sync_copy(alpha_hbm, alpha_v)
    alpha = alpha_v[...]
    lanes = lax.iota(jnp.int32, _L)

    def compute_block(srows, drows, erows, wo, i0, nvalid):
        rows = lanes + i0
        if nvalid < _L:
            rows = jnp.minimum(rows, nvalid - 1)

        def colgroup(g, acc):
            j0 = g * _L
            wv = watt_v[pl.ds(j0, _L)]
            for k in range(_L):
                jv = jnp.full((_L,), j0 + k, dtype=jnp.int32)
                sv = plsc.load_gather(srows, [rows, jv])
                dv = plsc.load_gather(drows, [rows, jv])
                ev = plsc.load_gather(erows, [rows, jv])
                z = sv + dv + ev
                p = jnp.maximum(z, 0.0) + alpha * jnp.minimum(z, 0.0)
                acc = acc + wv[k] * p
            return acc

        acc = lax.fori_loop(0, D // _L, colgroup, jnp.zeros((_L,), jnp.float32))
        wo[pl.ds(i0, _L)] = acc

    def chunk(g, _):
        b = base + g * _AC
        pltpu.sync_copy(src_hbm.at[pl.ds(b, _AC)], sidx)
        pltpu.sync_copy(dst_hbm.at[pl.ds(b, _AC)], didx)
        cp1 = pltpu.async_copy(hs_hbm.at[sidx], s_rows, sem_s)
        cp2 = pltpu.async_copy(hd_hbm.at[didx], d_rows, sem_d)
        pltpu.sync_copy(ew_hbm.at[pl.ds(b, _AC)], ew_rows)
        cp1.wait()
        cp2.wait()
        for i0 in range(0, _AC, _L):
            compute_block(s_rows, d_rows, ew_rows, wout, i0, _L)
        pltpu.sync_copy(wout, w_hbm.at[pl.ds(b, _AC)])
        return _

    lax.fori_loop(0, nchunk, chunk, 0)

    bt = base + nchunk * _AC
    pltpu.sync_copy(src_hbm.at[pl.ds(bt, _ATAIL)], sidx_t)
    pltpu.sync_copy(dst_hbm.at[pl.ds(bt, _ATAIL)], didx_t)
    cp1 = pltpu.async_copy(hs_hbm.at[sidx_t], s_rows_t, sem_s)
    cp2 = pltpu.async_copy(hd_hbm.at[didx_t], d_rows_t, sem_d)
    pltpu.sync_copy(ew_hbm.at[pl.ds(bt, _ATAIL)], ew_rows_t)
    cp1.wait()
    cp2.wait()
    compute_block(s_rows_t, d_rows_t, ew_rows_t, wout_t, 0, _ATAIL)
    pltpu.sync_copy(wout_t.at[pl.ds(0, _ATAIL)], w_hbm.at[pl.ds(bt, _ATAIL)])


def _logits_sc(hs, hd, ew, src, dst, watt, alpha):
    E = src.shape[0]
    D = hs.shape[1]
    mesh = plsc.VectorSubcoreMesh(core_axis_name="c", subcore_axis_name="s",
                                  num_cores=_NC, num_subcores=_NS)
    f = functools.partial(
        pl.kernel, _logits_body,
        out_type=jax.ShapeDtypeStruct((E,), jnp.float32),
        mesh=mesh,
        compiler_params=pltpu.CompilerParams(use_tc_tiling_on_sc=False, needs_layout_passes=False),
        scratch_types=[
            pltpu.VMEM((_AC,), jnp.int32),
            pltpu.VMEM((_AC,), jnp.int32),
            pltpu.VMEM((_AC, D), jnp.float32),
            pltpu.VMEM((_AC, D), jnp.float32),
            pltpu.VMEM((_AC, D), jnp.float32),
            pltpu.VMEM((_AC,), jnp.float32),
            pltpu.VMEM((_ATAIL,), jnp.int32),
            pltpu.VMEM((_ATAIL,), jnp.int32),
            pltpu.VMEM((_ATAIL, D), jnp.float32),
            pltpu.VMEM((_ATAIL, D), jnp.float32),
            pltpu.VMEM((_ATAIL, D), jnp.float32),
            pltpu.VMEM((_L,), jnp.float32),
            pltpu.VMEM((D,), jnp.float32),
            pltpu.VMEM((_L,), jnp.float32),
            pltpu.SemaphoreType.DMA,
            pltpu.SemaphoreType.DMA,
        ])()
    return f(hs, hd, ew, src, dst, watt, alpha)


def kernel(feat, edge_index, edge_weight, W_neigh, W_dst, W_self, W_edge, b_edge,
           W_prj_src, b_prj_src, W_prj_dst, b_prj_dst, W_prj_edge, b_prj_edge,
           W_att, b_att, prelu_alpha, out_bias):
    src = edge_index[0]
    dst = edge_index[1]
    n = feat.shape[0]
    d = feat.shape[1]

    Wn = jnp.concatenate([W_prj_src.T, W_prj_dst.T, W_dst.T, W_self.T], axis=1)
    bn = jnp.concatenate([b_prj_src, b_prj_dst,
                          jnp.zeros_like(b_prj_src), out_bias])[None, :]
    hw_src, hw_dst, h_dst, self_out = _mm4(feat, Wn, bn, 2000)

    We = jnp.concatenate([W_prj_edge.T, W_edge.T], axis=1)
    be = jnp.concatenate([b_prj_edge, b_edge])[None, :]
    ew, e = _mm2(edge_weight, We, be, 2000)

    watt = W_att[0]
    alpha16 = jnp.broadcast_to(prelu_alpha, (_L,)).astype(jnp.float32)
    w = _logits_sc(hw_src, hw_dst, ew, src, dst, watt, alpha16)

    m = jax.ops.segment_max(w, dst, num_segments=n)
    m = jnp.where(jnp.isfinite(m), m, 0.0)
    ex = jnp.exp(w - m[dst])
    ssum = jax.ops.segment_sum(ex, dst, num_segments=n)
    a = ex / ssum[dst]
    l = a[:, None] * e * feat[src]
    acc = jax.ops.segment_sum(l, dst, num_segments=n)

    return _final(self_out, h_dst, acc, W_neigh.T, 2000)


# speedup vs baseline: 1.3845x; 1.3845x over previous
import functools

import jax
import jax.numpy as jnp
from jax import lax
from jax.experimental import pallas as pl
from jax.experimental.pallas import tpu as pltpu
from jax.experimental.pallas import tpu_sc as plsc

_NC = 2
_NS = 16
_NW = _NC * _NS
_L = 16


def _mm_bias_body(x_ref, w_ref, b_ref, o_ref):
    o_ref[...] = jnp.dot(x_ref[...], w_ref[...],
                         preferred_element_type=jnp.float32) + b_ref[...]


def _mm(x, w, b, block_rows):
    M, K = x.shape
    _, Nc = w.shape
    return pl.pallas_call(
        _mm_bias_body,
        grid=(M // block_rows,),
        in_specs=[pl.BlockSpec((block_rows, K), lambda i: (i, 0)),
                  pl.BlockSpec((K, Nc), lambda i: (0, 0)),
                  pl.BlockSpec((1, Nc), lambda i: (0, 0))],
        out_specs=pl.BlockSpec((block_rows, Nc), lambda i: (i, 0)),
        out_shape=jax.ShapeDtypeStruct((M, Nc), jnp.float32),
    )(x, w, b)


def _mm4_body(x_ref, w_ref, b_ref, o1, o2, o3, o4):
    d = o1.shape[1]
    r = jnp.dot(x_ref[...], w_ref[...],
                preferred_element_type=jnp.float32) + b_ref[...]
    o1[...] = r[:, 0 * d:1 * d]
    o2[...] = r[:, 1 * d:2 * d]
    o3[...] = r[:, 2 * d:3 * d]
    o4[...] = r[:, 3 * d:4 * d]


def _mm4(x, w, b, block_rows):
    M, K = x.shape
    _, Nc = w.shape
    d = Nc // 4
    spec = pl.BlockSpec((block_rows, d), lambda i: (i, 0))
    return pl.pallas_call(
        _mm4_body,
        grid=(M // block_rows,),
        in_specs=[pl.BlockSpec((block_rows, K), lambda i: (i, 0)),
                  pl.BlockSpec((K, Nc), lambda i: (0, 0)),
                  pl.BlockSpec((1, Nc), lambda i: (0, 0))],
        out_specs=[spec, spec, spec, spec],
        out_shape=[jax.ShapeDtypeStruct((M, d), jnp.float32)] * 4,
    )(x, w, b)


def _mm2_body(x_ref, w_ref, b_ref, o1, o2):
    d = o1.shape[1]
    r = jnp.dot(x_ref[...], w_ref[...],
                preferred_element_type=jnp.float32) + b_ref[...]
    o1[...] = r[:, 0 * d:1 * d]
    o2[...] = r[:, 1 * d:2 * d]


def _mm2(x, w, b, block_rows):
    M, K = x.shape
    _, Nc = w.shape
    d = Nc // 2
    spec = pl.BlockSpec((block_rows, d), lambda i: (i, 0))
    return pl.pallas_call(
        _mm2_body,
        grid=(M // block_rows,),
        in_specs=[pl.BlockSpec((block_rows, K), lambda i: (i, 0)),
                  pl.BlockSpec((K, Nc), lambda i: (0, 0)),
                  pl.BlockSpec((1, Nc), lambda i: (0, 0))],
        out_specs=[spec, spec],
        out_shape=[jax.ShapeDtypeStruct((M, d), jnp.float32)] * 2,
    )(x, w, b)


def _final_body(s_ref, hd_ref, acc_ref, w_ref, o_ref):
    prod = hd_ref[...] * acc_ref[...]
    o_ref[...] = s_ref[...] + jnp.dot(prod, w_ref[...],
                                      preferred_element_type=jnp.float32)


def _final(self_out, h_dst, acc, w_neigh_t, block_rows):
    M, D = self_out.shape
    return pl.pallas_call(
        _final_body,
        grid=(M // block_rows,),
        in_specs=[pl.BlockSpec((block_rows, D), lambda i: (i, 0)),
                  pl.BlockSpec((block_rows, D), lambda i: (i, 0)),
                  pl.BlockSpec((block_rows, D), lambda i: (i, 0)),
                  pl.BlockSpec((D, D), lambda i: (0, 0))],
        out_specs=pl.BlockSpec((block_rows, D), lambda i: (i, 0)),
        out_shape=jax.ShapeDtypeStruct((M, D), jnp.float32),
    )(self_out, h_dst, acc, w_neigh_t)


_AC = 128
_ATAIL = 8


def _logits_body(hs_hbm, hd_hbm, ew_hbm, src_hbm, dst_hbm, watt_hbm, alpha_hbm,
                 w_hbm,
                 sidx, didx, s_rows, d_rows, ew_rows, wout,
                 sidx_t, didx_t, s_rows_t, d_rows_t, ew_rows_t, wout_t,
                 watt_v, alpha_v, sem_s, sem_d):
    D = hs_hbm.shape[1]
    E = src_hbm.shape[0]
    epw = E // _NW
    nchunk = (epw - _ATAIL) // _AC
    c = lax.axis_index("c")
    s = lax.axis_index("s")
    wid = s * _NC + c
    base = wid * epw

    pltpu.sync_copy(watt_hbm, watt_v)
    pltpu.sync_copy(alpha_hbm, alpha_v)
    alpha = alpha_v[...]
    lanes = lax.iota(jnp.int32, _L)

    nsub = D // _L
    wsubs = [watt_v[pl.ds(j0 * _L, _L)] for j0 in range(nsub)]

    def compute_group(srows, drows, erows, wo, gi, nvalid):
        def edge(k, wvec):
            i = gi * _L + k
            if nvalid < _L:
                i = jnp.minimum(i, nvalid - 1)
            acc = jnp.zeros((_L,), jnp.float32)
            for j0 in range(nsub):
                sl = pl.ds(j0 * _L, _L)
                z = srows[i, sl] + drows[i, sl] + erows[i, sl]
                p = jnp.maximum(z, 0.0) + alpha * jnp.minimum(z, 0.0)
                acc = acc + wsubs[j0] * p
            tot = jnp.sum(acc)
            return jnp.where(lanes == k, tot, wvec)

        wvec = lax.fori_loop(0, _L, edge, jnp.zeros((_L,), jnp.float32))
        wo[pl.ds(gi * _L, _L)] = wvec

    def chunk(g, _):
        b = base + g * _AC
        pltpu.sync_copy(src_hbm.at[pl.ds(b, _AC)], sidx)
        pltpu.sync_copy(dst_hbm.at[pl.ds(b, _AC)], didx)
        cp1 = pltpu.async_copy(hs_hbm.at[sidx], s_rows, sem_s)
        cp2 = pltpu.async_copy(hd_hbm.at[didx], d_rows, sem_d)
        pltpu.sync_copy(ew_hbm.at[pl.ds(b, _AC)], ew_rows)
        cp1.wait()
        cp2.wait()
        def grp(gi, _u):
            compute_group(s_rows, d_rows, ew_rows, wout, gi, _L)
            return _u
        lax.fori_loop(0, _AC // _L, grp, 0)
        pltpu.sync_copy(wout, w_hbm.at[pl.ds(b, _AC)])
        return _

    lax.fori_loop(0, nchunk, chunk, 0)

    bt = base + nchunk * _AC
    pltpu.sync_copy(src_hbm.at[pl.ds(bt, _ATAIL)], sidx_t)
    pltpu.sync_copy(dst_hbm.at[pl.ds(bt, _ATAIL)], didx_t)
    cp1 = pltpu.async_copy(hs_hbm.at[sidx_t], s_rows_t, sem_s)
    cp2 = pltpu.async_copy(hd_hbm.at[didx_t], d_rows_t, sem_d)
    pltpu.sync_copy(ew_hbm.at[pl.ds(bt, _ATAIL)], ew_rows_t)
    cp1.wait()
    cp2.wait()
    compute_group(s_rows_t, d_rows_t, ew_rows_t, wout_t, 0, _ATAIL)
    pltpu.sync_copy(wout_t.at[pl.ds(0, _ATAIL)], w_hbm.at[pl.ds(bt, _ATAIL)])


def _logits_sc(hs, hd, ew, src, dst, watt, alpha):
    E = src.shape[0]
    D = hs.shape[1]
    mesh = plsc.VectorSubcoreMesh(core_axis_name="c", subcore_axis_name="s",
                                  num_cores=_NC, num_subcores=_NS)
    f = functools.partial(
        pl.kernel, _logits_body,
        out_type=jax.ShapeDtypeStruct((E,), jnp.float32),
        mesh=mesh,
        compiler_params=pltpu.CompilerParams(use_tc_tiling_on_sc=False, needs_layout_passes=False),
        scratch_types=[
            pltpu.VMEM((_AC,), jnp.int32),
            pltpu.VMEM((_AC,), jnp.int32),
            pltpu.VMEM((_AC, D), jnp.float32),
            pltpu.VMEM((_AC, D), jnp.float32),
            pltpu.VMEM((_AC, D), jnp.float32),
            pltpu.VMEM((_AC,), jnp.float32),
            pltpu.VMEM((_ATAIL,), jnp.int32),
            pltpu.VMEM((_ATAIL,), jnp.int32),
            pltpu.VMEM((_ATAIL, D), jnp.float32),
            pltpu.VMEM((_ATAIL, D), jnp.float32),
            pltpu.VMEM((_ATAIL, D), jnp.float32),
            pltpu.VMEM((_L,), jnp.float32),
            pltpu.VMEM((D,), jnp.float32),
            pltpu.VMEM((_L,), jnp.float32),
            pltpu.SemaphoreType.DMA,
            pltpu.SemaphoreType.DMA,
        ])()
    return f(hs, hd, ew, src, dst, watt, alpha)


def kernel(feat, edge_index, edge_weight, W_neigh, W_dst, W_self, W_edge, b_edge,
           W_prj_src, b_prj_src, W_prj_dst, b_prj_dst, W_prj_edge, b_prj_edge,
           W_att, b_att, prelu_alpha, out_bias):
    src = edge_index[0]
    dst = edge_index[1]
    n = feat.shape[0]
    d = feat.shape[1]

    Wn = jnp.concatenate([W_prj_src.T, W_prj_dst.T, W_dst.T, W_self.T], axis=1)
    bn = jnp.concatenate([b_prj_src, b_prj_dst,
                          jnp.zeros_like(b_prj_src), out_bias])[None, :]
    hw_src, hw_dst, h_dst, self_out = _mm4(feat, Wn, bn, 2000)

    We = jnp.concatenate([W_prj_edge.T, W_edge.T], axis=1)
    be = jnp.concatenate([b_prj_edge, b_edge])[None, :]
    ew, e = _mm2(edge_weight, We, be, 2000)

    watt = W_att[0]
    alpha16 = jnp.broadcast_to(prelu_alpha, (_L,)).astype(jnp.float32)
    w = _logits_sc(hw_src, hw_dst, ew, src, dst, watt, alpha16)

    m = jax.ops.segment_max(w, dst, num_segments=n)
    m = jnp.where(jnp.isfinite(m), m, 0.0)
    ex = jnp.exp(w - m[dst])
    ssum = jax.ops.segment_sum(ex, dst, num_segments=n)
    a = ex / ssum[dst]
    l = a[:, None] * e * feat[src]
    acc = jax.ops.segment_sum(l, dst, num_segments=n)

    return _final(self_out, h_dst, acc, W_neigh.T, 2000)


# speedup vs baseline: 1.7056x; 1.2319x over previous
import functools

import jax
import jax.numpy as jnp
from jax import lax
from jax.experimental import pallas as pl
from jax.experimental.pallas import tpu as pltpu
from jax.experimental.pallas import tpu_sc as plsc

_NC = 2
_NS = 16
_NW = _NC * _NS
_L = 16


def _mm_bias_body(x_ref, w_ref, b_ref, o_ref):
    o_ref[...] = jnp.dot(x_ref[...], w_ref[...],
                         preferred_element_type=jnp.float32) + b_ref[...]


def _mm(x, w, b, block_rows):
    M, K = x.shape
    _, Nc = w.shape
    return pl.pallas_call(
        _mm_bias_body,
        grid=(M // block_rows,),
        in_specs=[pl.BlockSpec((block_rows, K), lambda i: (i, 0)),
                  pl.BlockSpec((K, Nc), lambda i: (0, 0)),
                  pl.BlockSpec((1, Nc), lambda i: (0, 0))],
        out_specs=pl.BlockSpec((block_rows, Nc), lambda i: (i, 0)),
        out_shape=jax.ShapeDtypeStruct((M, Nc), jnp.float32),
    )(x, w, b)


def _mm4_body(x_ref, w_ref, b_ref, o1, o2, o3, o4):
    d = o1.shape[1]
    r = jnp.dot(x_ref[...], w_ref[...],
                preferred_element_type=jnp.float32) + b_ref[...]
    o1[...] = r[:, 0 * d:1 * d]
    o2[...] = r[:, 1 * d:2 * d]
    o3[...] = r[:, 2 * d:3 * d]
    o4[...] = r[:, 3 * d:4 * d]


def _mm4(x, w, b, block_rows):
    M, K = x.shape
    _, Nc = w.shape
    d = Nc // 4
    spec = pl.BlockSpec((block_rows, d), lambda i: (i, 0))
    return pl.pallas_call(
        _mm4_body,
        grid=(M // block_rows,),
        in_specs=[pl.BlockSpec((block_rows, K), lambda i: (i, 0)),
                  pl.BlockSpec((K, Nc), lambda i: (0, 0)),
                  pl.BlockSpec((1, Nc), lambda i: (0, 0))],
        out_specs=[spec, spec, spec, spec],
        out_shape=[jax.ShapeDtypeStruct((M, d), jnp.float32)] * 4,
    )(x, w, b)


def _mm2_body(x_ref, w_ref, b_ref, o1, o2):
    d = o1.shape[1]
    r = jnp.dot(x_ref[...], w_ref[...],
                preferred_element_type=jnp.float32) + b_ref[...]
    o1[...] = r[:, 0 * d:1 * d]
    o2[...] = r[:, 1 * d:2 * d]


def _mm2(x, w, b, block_rows):
    M, K = x.shape
    _, Nc = w.shape
    d = Nc // 2
    spec = pl.BlockSpec((block_rows, d), lambda i: (i, 0))
    return pl.pallas_call(
        _mm2_body,
        grid=(M // block_rows,),
        in_specs=[pl.BlockSpec((block_rows, K), lambda i: (i, 0)),
                  pl.BlockSpec((K, Nc), lambda i: (0, 0)),
                  pl.BlockSpec((1, Nc), lambda i: (0, 0))],
        out_specs=[spec, spec],
        out_shape=[jax.ShapeDtypeStruct((M, d), jnp.float32)] * 2,
    )(x, w, b)


def _final_body(s_ref, hd_ref, acc_ref, w_ref, o_ref):
    prod = hd_ref[...] * acc_ref[...]
    o_ref[...] = s_ref[...] + jnp.dot(prod, w_ref[...],
                                      preferred_element_type=jnp.float32)


def _final(self_out, h_dst, acc, w_neigh_t, block_rows):
    M, D = self_out.shape
    return pl.pallas_call(
        _final_body,
        grid=(M // block_rows,),
        in_specs=[pl.BlockSpec((block_rows, D), lambda i: (i, 0)),
                  pl.BlockSpec((block_rows, D), lambda i: (i, 0)),
                  pl.BlockSpec((block_rows, D), lambda i: (i, 0)),
                  pl.BlockSpec((D, D), lambda i: (0, 0))],
        out_specs=pl.BlockSpec((block_rows, D), lambda i: (i, 0)),
        out_shape=jax.ShapeDtypeStruct((M, D), jnp.float32),
    )(self_out, h_dst, acc, w_neigh_t)


_AC = 128
_ATAIL = 8


def _logits_body(hs_hbm, hd_hbm, ew_hbm, src_hbm, dst_hbm, watt_hbm, alpha_hbm,
                 w_hbm,
                 sidx, didx, s_rows, d_rows, ew_rows, wout,
                 sidx_t, didx_t, s_rows_t, d_rows_t, ew_rows_t, wout_t,
                 watt_v, alpha_v, sem_s, sem_d):
    D = hs_hbm.shape[1]
    E = src_hbm.shape[0]
    epw = E // _NW
    nchunk = (epw - _ATAIL) // _AC
    c = lax.axis_index("c")
    s = lax.axis_index("s")
    wid = s * _NC + c
    base = wid * epw

    pltpu.sync_copy(watt_hbm, watt_v)
    pltpu.sync_copy(alpha_hbm, alpha_v)
    alpha = alpha_v[...]
    lanes = lax.iota(jnp.int32, _L)

    nsub = D // _L
    wsubs = [watt_v[pl.ds(j0 * _L, _L)] for j0 in range(nsub)]

    def compute_group(srows, drows, erows, wo, gi, nvalid):
        def edge(k, wvec):
            i = gi * _L + k
            if nvalid < _L:
                i = jnp.minimum(i, nvalid - 1)
            acc = jnp.zeros((_L,), jnp.float32)
            for j0 in range(nsub):
                sl = pl.ds(j0 * _L, _L)
                z = srows[i, sl] + drows[i, sl] + erows[i, sl]
                p = jnp.maximum(z, 0.0) + alpha * jnp.minimum(z, 0.0)
                acc = acc + wsubs[j0] * p
            tot = jnp.sum(acc)
            return jnp.where(lanes == k, tot, wvec)

        wvec = lax.fori_loop(0, _L, edge, jnp.zeros((_L,), jnp.float32))
        wo[pl.ds(gi * _L, _L)] = wvec

    def chunk(g, _):
        b = base + g * _AC
        pltpu.sync_copy(src_hbm.at[pl.ds(b, _AC)], sidx)
        pltpu.sync_copy(dst_hbm.at[pl.ds(b, _AC)], didx)
        cp1 = pltpu.async_copy(hs_hbm.at[sidx], s_rows, sem_s)
        cp2 = pltpu.async_copy(hd_hbm.at[didx], d_rows, sem_d)
        pltpu.sync_copy(ew_hbm.at[pl.ds(b, _AC)], ew_rows)
        cp1.wait()
        cp2.wait()
        def grp(gi, _u):
            compute_group(s_rows, d_rows, ew_rows, wout, gi, _L)
            return _u
        lax.fori_loop(0, _AC // _L, grp, 0)
        pltpu.sync_copy(wout, w_hbm.at[pl.ds(b, _AC)])
        return _

    lax.fori_loop(0, nchunk, chunk, 0)

    bt = base + nchunk * _AC
    pltpu.sync_copy(src_hbm.at[pl.ds(bt, _ATAIL)], sidx_t)
    pltpu.sync_copy(dst_hbm.at[pl.ds(bt, _ATAIL)], didx_t)
    cp1 = pltpu.async_copy(hs_hbm.at[sidx_t], s_rows_t, sem_s)
    cp2 = pltpu.async_copy(hd_hbm.at[didx_t], d_rows_t, sem_d)
    pltpu.sync_copy(ew_hbm.at[pl.ds(bt, _ATAIL)], ew_rows_t)
    cp1.wait()
    cp2.wait()
    compute_group(s_rows_t, d_rows_t, ew_rows_t, wout_t, 0, _ATAIL)
    pltpu.sync_copy(wout_t.at[pl.ds(0, _ATAIL)], w_hbm.at[pl.ds(bt, _ATAIL)])


def _logits_sc(hs, hd, ew, src, dst, watt, alpha):
    E = src.shape[0]
    D = hs.shape[1]
    mesh = plsc.VectorSubcoreMesh(core_axis_name="c", subcore_axis_name="s",
                                  num_cores=_NC, num_subcores=_NS)
    f = functools.partial(
        pl.kernel, _logits_body,
        out_type=jax.ShapeDtypeStruct((E,), jnp.float32),
        mesh=mesh,
        compiler_params=pltpu.CompilerParams(use_tc_tiling_on_sc=False, needs_layout_passes=False),
        scratch_types=[
            pltpu.VMEM((_AC,), jnp.int32),
            pltpu.VMEM((_AC,), jnp.int32),
            pltpu.VMEM((_AC, D), jnp.float32),
            pltpu.VMEM((_AC, D), jnp.float32),
            pltpu.VMEM((_AC, D), jnp.float32),
            pltpu.VMEM((_AC,), jnp.float32),
            pltpu.VMEM((_ATAIL,), jnp.int32),
            pltpu.VMEM((_ATAIL,), jnp.int32),
            pltpu.VMEM((_ATAIL, D), jnp.float32),
            pltpu.VMEM((_ATAIL, D), jnp.float32),
            pltpu.VMEM((_ATAIL, D), jnp.float32),
            pltpu.VMEM((_L,), jnp.float32),
            pltpu.VMEM((D,), jnp.float32),
            pltpu.VMEM((_L,), jnp.float32),
            pltpu.SemaphoreType.DMA,
            pltpu.SemaphoreType.DMA,
        ])()
    return f(hs, hd, ew, src, dst, watt, alpha)


_NEG = -3.4e38


def _tile_slices(s):
    return s * 624


def _segmax_body(w_hbm, dst_hbm, m2_hbm, w_v, dst_v, m_loc, mrows, mout, sp_ref):
    E = w_hbm.shape[0]
    N = m2_hbm.shape[1]
    epw = E // _NW
    ngrp = (epw + _L - 1) // _L
    c = lax.axis_index("c")
    s = lax.axis_index("s")
    wid = s * _NC + c
    base = wid * epw
    lanes = lax.iota(jnp.int32, _L)

    pltpu.sync_copy(w_hbm.at[pl.ds(base, epw)], w_v.at[pl.ds(0, epw)])
    pltpu.sync_copy(dst_hbm.at[pl.ds(base, epw)], dst_v.at[pl.ds(0, epw)])

    def initloop(i, _u):
        m_loc[pl.ds(i * _L, _L)] = jnp.full((_L,), _NEG, jnp.float32)
        return _u
    lax.fori_loop(0, (N + _L) // _L, initloop, 0)

    def grp(gi, _u):
        valid = (gi * _L + lanes) < epw
        dvec = jnp.where(valid, dst_v[pl.ds(gi * _L, _L)], 0)
        wvec = jnp.where(valid, w_v[pl.ds(gi * _L, _L)], _NEG)
        for k in range(_L):
            d = dvec[k]
            x = wvec[k]
            cur = m_loc[pl.ds(d, _L)]
            m_loc[pl.ds(d, _L)] = jnp.where(lanes == 0,
                                            jnp.maximum(cur, x), cur)
        return _u
    lax.fori_loop(0, ngrp, grp, 0)

    if True:
        pltpu.sync_copy(m_loc.at[pl.ds(0, N)], sp_ref.at[s])
        plsc.subcore_barrier()
        off = _tile_slices(s)
        pltpu.sync_copy(sp_ref.at[:, pl.ds(off, 640)], mrows)

        def redgrp(j, _u):
            acc = mrows[0, pl.ds(j * _L, _L)]
            for r in range(1, _NS):
                acc = jnp.maximum(acc, mrows[r, pl.ds(j * _L, _L)])
            mout[pl.ds(j * _L, _L)] = acc
            return _u
        lax.fori_loop(0, 640 // _L, redgrp, 0)

        @pl.when(s == _NS - 1)
        def _():
            pltpu.sync_copy(mout, m2_hbm.at[c, pl.ds(off, 640)])

        @pl.when(s < _NS - 1)
        def _():
            pltpu.sync_copy(mout.at[pl.ds(0, 624)],
                            m2_hbm.at[c, pl.ds(off, 624)])


def _segmax_sc(w, dst, n):
    E = w.shape[0]
    epw = E // _NW
    mesh = plsc.VectorSubcoreMesh(core_axis_name="c", subcore_axis_name="s",
                                  num_cores=_NC, num_subcores=_NS)
    f = functools.partial(
        pl.kernel, _segmax_body,
        out_type=jax.ShapeDtypeStruct((_NC, n), jnp.float32),
        mesh=mesh,
        compiler_params=pltpu.CompilerParams(use_tc_tiling_on_sc=False,
                                             needs_layout_passes=False),
        scratch_types=[
            pltpu.VMEM((epw + _L,), jnp.float32),
            pltpu.VMEM((epw + _L,), jnp.int32),
            pltpu.VMEM((n + _L,), jnp.float32),
            pltpu.VMEM((_NS, 640), jnp.float32),
            pltpu.VMEM((640,), jnp.float32),
            pltpu.VMEM_SHARED((_NS, n), jnp.float32),
        ])()
    return f(w, dst)


def _expsum_body(w_hbm, dst_hbm, m2_hbm, ex_hbm, ss2_hbm,
                 w_v, dst_v, ex_v, m_loc, ss_loc, m2v, mrows, mout, sp_ref):
    E = w_hbm.shape[0]
    N = m2_hbm.shape[1]
    epw = E // _NW
    ngrp = (epw + _L - 1) // _L
    c = lax.axis_index("c")
    s = lax.axis_index("s")
    wid = s * _NC + c
    base = wid * epw
    lanes = lax.iota(jnp.int32, _L)

    pltpu.sync_copy(w_hbm.at[pl.ds(base, epw)], w_v.at[pl.ds(0, epw)])
    pltpu.sync_copy(dst_hbm.at[pl.ds(base, epw)], dst_v.at[pl.ds(0, epw)])
    pltpu.sync_copy(m2_hbm, m2v)

    def initloop(i, _u):
        sl = pl.ds(i * _L, _L)
        ss_loc[sl] = jnp.zeros((_L,), jnp.float32)
        return _u
    lax.fori_loop(0, (N + _L) // _L, initloop, 0)

    def mloop(i, _u):
        sl = pl.ds(i * _L, _L)
        m_loc[sl] = jnp.maximum(m2v[0, sl], m2v[1, sl])
        return _u
    lax.fori_loop(0, N // _L, mloop, 0)

    def grp(gi, _u):
        valid = (gi * _L + lanes) < epw
        sl = pl.ds(gi * _L, _L)
        dvec = jnp.where(valid, dst_v[sl], 0)
        wvec = w_v[sl]
        mg = plsc.load_gather(m_loc, [dvec])
        exv = jnp.where(valid, jnp.exp(wvec - mg), 0.0)
        ex_v[sl] = exv
        for k in range(_L):
            d = dvec[k]
            x = exv[k]
            cur = ss_loc[pl.ds(d, _L)]
            ss_loc[pl.ds(d, _L)] = jnp.where(lanes == 0, cur + x, cur)
        return _u
    lax.fori_loop(0, ngrp, grp, 0)

    pltpu.sync_copy(ex_v.at[pl.ds(0, epw)], ex_hbm.at[pl.ds(base, epw)])

    if True:
        pltpu.sync_copy(ss_loc.at[pl.ds(0, N)], sp_ref.at[s])
        plsc.subcore_barrier()
        off = _tile_slices(s)
        pltpu.sync_copy(sp_ref.at[:, pl.ds(off, 640)], mrows)

        def redgrp(j, _u):
            acc = mrows[0, pl.ds(j * _L, _L)]
            for r in range(1, _NS):
                acc = acc + mrows[r, pl.ds(j * _L, _L)]
            mout[pl.ds(j * _L, _L)] = acc
            return _u
        lax.fori_loop(0, 640 // _L, redgrp, 0)

        @pl.when(s == _NS - 1)
        def _():
            pltpu.sync_copy(mout, ss2_hbm.at[c, pl.ds(off, 640)])

        @pl.when(s < _NS - 1)
        def _():
            pltpu.sync_copy(mout.at[pl.ds(0, 624)],
                            ss2_hbm.at[c, pl.ds(off, 624)])


def _expsum_sc(w, dst, m2, n):
    E = w.shape[0]
    epw = E // _NW
    mesh = plsc.VectorSubcoreMesh(core_axis_name="c", subcore_axis_name="s",
                                  num_cores=_NC, num_subcores=_NS)
    f = functools.partial(
        pl.kernel, _expsum_body,
        out_type=[jax.ShapeDtypeStruct((E,), jnp.float32),
                  jax.ShapeDtypeStruct((_NC, n), jnp.float32)],
        mesh=mesh,
        compiler_params=pltpu.CompilerParams(use_tc_tiling_on_sc=False,
                                             needs_layout_passes=False),
        scratch_types=[
            pltpu.VMEM((epw + _L,), jnp.float32),
            pltpu.VMEM((epw + _L,), jnp.int32),
            pltpu.VMEM((epw + _L,), jnp.float32),
            pltpu.VMEM((n + _L,), jnp.float32),
            pltpu.VMEM((n + _L,), jnp.float32),
            pltpu.VMEM((_NC, n), jnp.float32),
            pltpu.VMEM((_NS, 640), jnp.float32),
            pltpu.VMEM((640,), jnp.float32),
            pltpu.VMEM_SHARED((_NS, n), jnp.float32),
        ])()
    return f(w, dst, m2)


def kernel(feat, edge_index, edge_weight, W_neigh, W_dst, W_self, W_edge, b_edge,
           W_prj_src, b_prj_src, W_prj_dst, b_prj_dst, W_prj_edge, b_prj_edge,
           W_att, b_att, prelu_alpha, out_bias):
    src = edge_index[0]
    dst = edge_index[1]
    n = feat.shape[0]
    d = feat.shape[1]

    Wn = jnp.concatenate([W_prj_src.T, W_prj_dst.T, W_dst.T, W_self.T], axis=1)
    bn = jnp.concatenate([b_prj_src, b_prj_dst,
                          jnp.zeros_like(b_prj_src), out_bias])[None, :]
    hw_src, hw_dst, h_dst, self_out = _mm4(feat, Wn, bn, 2000)

    We = jnp.concatenate([W_prj_edge.T, W_edge.T], axis=1)
    be = jnp.concatenate([b_prj_edge, b_edge])[None, :]
    ew, e = _mm2(edge_weight, We, be, 2000)

    watt = W_att[0]
    alpha16 = jnp.broadcast_to(prelu_alpha, (_L,)).astype(jnp.float32)
    w = _logits_sc(hw_src, hw_dst, ew, src, dst, watt, alpha16)

    m2 = _segmax_sc(w, dst, n)
    ex, ss2 = _expsum_sc(w, dst, m2, n)

    ssum = ss2[0] + ss2[1]
    a = ex / ssum[dst]
    l = a[:, None] * e * feat[src]
    acc = jax.ops.segment_sum(l, dst, num_segments=n)

    return _final(self_out, h_dst, acc, W_neigh.T, 2000)


# speedup vs baseline: 3.1892x; 1.8698x over previous
import functools

import jax
import jax.numpy as jnp
from jax import lax
from jax.experimental import pallas as pl
from jax.experimental.pallas import tpu as pltpu
from jax.experimental.pallas import tpu_sc as plsc

_NC = 2
_NS = 16
_NW = _NC * _NS
_L = 16


def _mm_bias_body(x_ref, w_ref, b_ref, o_ref):
    o_ref[...] = jnp.dot(x_ref[...], w_ref[...],
                         preferred_element_type=jnp.float32) + b_ref[...]


def _mm(x, w, b, block_rows):
    M, K = x.shape
    _, Nc = w.shape
    return pl.pallas_call(
        _mm_bias_body,
        grid=(M // block_rows,),
        in_specs=[pl.BlockSpec((block_rows, K), lambda i: (i, 0)),
                  pl.BlockSpec((K, Nc), lambda i: (0, 0)),
                  pl.BlockSpec((1, Nc), lambda i: (0, 0))],
        out_specs=pl.BlockSpec((block_rows, Nc), lambda i: (i, 0)),
        out_shape=jax.ShapeDtypeStruct((M, Nc), jnp.float32),
    )(x, w, b)


def _mm4_body(x_ref, w_ref, b_ref, o1, o2, o3, o4):
    d = o1.shape[1]
    r = jnp.dot(x_ref[...], w_ref[...],
                preferred_element_type=jnp.float32) + b_ref[...]
    o1[...] = r[:, 0 * d:1 * d]
    o2[...] = r[:, 1 * d:2 * d]
    o3[...] = r[:, 2 * d:3 * d]
    o4[...] = r[:, 3 * d:4 * d]


def _mm4(x, w, b, block_rows):
    M, K = x.shape
    _, Nc = w.shape
    d = Nc // 4
    spec = pl.BlockSpec((block_rows, d), lambda i: (i, 0))
    return pl.pallas_call(
        _mm4_body,
        grid=(M // block_rows,),
        in_specs=[pl.BlockSpec((block_rows, K), lambda i: (i, 0)),
                  pl.BlockSpec((K, Nc), lambda i: (0, 0)),
                  pl.BlockSpec((1, Nc), lambda i: (0, 0))],
        out_specs=[spec, spec, spec, spec],
        out_shape=[jax.ShapeDtypeStruct((M, d), jnp.float32)] * 4,
    )(x, w, b)


def _mm2_body(x_ref, w_ref, b_ref, o1, o2):
    d = o1.shape[1]
    h = d // 2
    r = jnp.dot(x_ref[...], w_ref[...],
                preferred_element_type=jnp.float32) + b_ref[...]
    o1[...] = r[:, 0 * d:1 * d]
    o2[0] = r[:, d:d + h]
    o2[1] = r[:, d + h:2 * d]


def _mm2(x, w, b, block_rows):
    M, K = x.shape
    _, Nc = w.shape
    d = Nc // 2
    h = d // 2
    return pl.pallas_call(
        _mm2_body,
        grid=(M // block_rows,),
        in_specs=[pl.BlockSpec((block_rows, K), lambda i: (i, 0)),
                  pl.BlockSpec((K, Nc), lambda i: (0, 0)),
                  pl.BlockSpec((1, Nc), lambda i: (0, 0))],
        out_specs=[pl.BlockSpec((block_rows, d), lambda i: (i, 0)),
                   pl.BlockSpec((2, block_rows, h), lambda i: (0, i, 0))],
        out_shape=[jax.ShapeDtypeStruct((M, d), jnp.float32),
                   jax.ShapeDtypeStruct((2, M, h), jnp.float32)],
    )(x, w, b)


def _final_body(s_ref, hd_ref, acc_ref, w_ref, o_ref):
    prod = hd_ref[...] * acc_ref[...]
    o_ref[...] = s_ref[...] + jnp.dot(prod, w_ref[...],
                                      preferred_element_type=jnp.float32)


def _final(self_out, h_dst, acc, w_neigh_t, block_rows):
    M, D = self_out.shape
    return pl.pallas_call(
        _final_body,
        grid=(M // block_rows,),
        in_specs=[pl.BlockSpec((block_rows, D), lambda i: (i, 0)),
                  pl.BlockSpec((block_rows, D), lambda i: (i, 0)),
                  pl.BlockSpec((block_rows, D), lambda i: (i, 0)),
                  pl.BlockSpec((D, D), lambda i: (0, 0))],
        out_specs=pl.BlockSpec((block_rows, D), lambda i: (i, 0)),
        out_shape=jax.ShapeDtypeStruct((M, D), jnp.float32),
    )(self_out, h_dst, acc, w_neigh_t)


_AC = 128
_ATAIL = 8


def _logits_body(hs_hbm, hd_hbm, ew_hbm, src_hbm, dst_hbm, watt_hbm, alpha_hbm,
                 w_hbm,
                 sidx, didx, s_rows, d_rows, ew_rows, wout,
                 sidx_t, didx_t, s_rows_t, d_rows_t, ew_rows_t, wout_t,
                 watt_v, alpha_v, sem_s, sem_d):
    D = hs_hbm.shape[1]
    E = src_hbm.shape[0]
    epw = E // _NW
    nchunk = (epw - _ATAIL) // _AC
    c = lax.axis_index("c")
    s = lax.axis_index("s")
    wid = s * _NC + c
    base = wid * epw

    pltpu.sync_copy(watt_hbm, watt_v)
    pltpu.sync_copy(alpha_hbm, alpha_v)
    alpha = alpha_v[...]
    lanes = lax.iota(jnp.int32, _L)

    nsub = D // _L
    wsubs = [watt_v[pl.ds(j0 * _L, _L)] for j0 in range(nsub)]

    def compute_group(srows, drows, erows, wo, gi, nvalid):
        def edge(k, wvec):
            i = gi * _L + k
            if nvalid < _L:
                i = jnp.minimum(i, nvalid - 1)
            acc = jnp.zeros((_L,), jnp.float32)
            for j0 in range(nsub):
                sl = pl.ds(j0 * _L, _L)
                z = srows[i, sl] + drows[i, sl] + erows[i, sl]
                p = jnp.maximum(z, 0.0) + alpha * jnp.minimum(z, 0.0)
                acc = acc + wsubs[j0] * p
            tot = jnp.sum(acc)
            return jnp.where(lanes == k, tot, wvec)

        wvec = lax.fori_loop(0, _L, edge, jnp.zeros((_L,), jnp.float32))
        wo[pl.ds(gi * _L, _L)] = wvec

    def chunk(g, _):
        b = base + g * _AC
        pltpu.sync_copy(src_hbm.at[pl.ds(b, _AC)], sidx)
        pltpu.sync_copy(dst_hbm.at[pl.ds(b, _AC)], didx)
        cp1 = pltpu.async_copy(hs_hbm.at[sidx], s_rows, sem_s)
        cp2 = pltpu.async_copy(hd_hbm.at[didx], d_rows, sem_d)
        pltpu.sync_copy(ew_hbm.at[pl.ds(b, _AC)], ew_rows)
        cp1.wait()
        cp2.wait()
        def grp(gi, _u):
            compute_group(s_rows, d_rows, ew_rows, wout, gi, _L)
            return _u
        lax.fori_loop(0, _AC // _L, grp, 0)
        pltpu.sync_copy(wout, w_hbm.at[pl.ds(b, _AC)])
        return _

    lax.fori_loop(0, nchunk, chunk, 0)

    bt = base + nchunk * _AC
    pltpu.sync_copy(src_hbm.at[pl.ds(bt, _ATAIL)], sidx_t)
    pltpu.sync_copy(dst_hbm.at[pl.ds(bt, _ATAIL)], didx_t)
    cp1 = pltpu.async_copy(hs_hbm.at[sidx_t], s_rows_t, sem_s)
    cp2 = pltpu.async_copy(hd_hbm.at[didx_t], d_rows_t, sem_d)
    pltpu.sync_copy(ew_hbm.at[pl.ds(bt, _ATAIL)], ew_rows_t)
    cp1.wait()
    cp2.wait()
    compute_group(s_rows_t, d_rows_t, ew_rows_t, wout_t, 0, _ATAIL)
    pltpu.sync_copy(wout_t.at[pl.ds(0, _ATAIL)], w_hbm.at[pl.ds(bt, _ATAIL)])


def _logits_sc(hs, hd, ew, src, dst, watt, alpha):
    E = src.shape[0]
    D = hs.shape[1]
    mesh = plsc.VectorSubcoreMesh(core_axis_name="c", subcore_axis_name="s",
                                  num_cores=_NC, num_subcores=_NS)
    f = functools.partial(
        pl.kernel, _logits_body,
        out_type=jax.ShapeDtypeStruct((E,), jnp.float32),
        mesh=mesh,
        compiler_params=pltpu.CompilerParams(use_tc_tiling_on_sc=False, needs_layout_passes=False),
        scratch_types=[
            pltpu.VMEM((_AC,), jnp.int32),
            pltpu.VMEM((_AC,), jnp.int32),
            pltpu.VMEM((_AC, D), jnp.float32),
            pltpu.VMEM((_AC, D), jnp.float32),
            pltpu.VMEM((_AC, D), jnp.float32),
            pltpu.VMEM((_AC,), jnp.float32),
            pltpu.VMEM((_ATAIL,), jnp.int32),
            pltpu.VMEM((_ATAIL,), jnp.int32),
            pltpu.VMEM((_ATAIL, D), jnp.float32),
            pltpu.VMEM((_ATAIL, D), jnp.float32),
            pltpu.VMEM((_ATAIL, D), jnp.float32),
            pltpu.VMEM((_L,), jnp.float32),
            pltpu.VMEM((D,), jnp.float32),
            pltpu.VMEM((_L,), jnp.float32),
            pltpu.SemaphoreType.DMA,
            pltpu.SemaphoreType.DMA,
        ])()
    return f(hs, hd, ew, src, dst, watt, alpha)


_NEG = -3.4e38


def _tile_slices(s):
    return s * 624


def _segmax_body(w_hbm, dst_hbm, m2_hbm, w_v, dst_v, m_loc, mrows, mout, sp_ref):
    E = w_hbm.shape[0]
    N = m2_hbm.shape[1]
    epw = E // _NW
    ngrp = (epw + _L - 1) // _L
    c = lax.axis_index("c")
    s = lax.axis_index("s")
    wid = s * _NC + c
    base = wid * epw
    lanes = lax.iota(jnp.int32, _L)

    pltpu.sync_copy(w_hbm.at[pl.ds(base, epw)], w_v.at[pl.ds(0, epw)])
    pltpu.sync_copy(dst_hbm.at[pl.ds(base, epw)], dst_v.at[pl.ds(0, epw)])

    def initloop(i, _u):
        m_loc[pl.ds(i * _L, _L)] = jnp.full((_L,), _NEG, jnp.float32)
        return _u
    lax.fori_loop(0, (N + _L) // _L, initloop, 0)

    def grp(gi, _u):
        valid = (gi * _L + lanes) < epw
        dvec = jnp.where(valid, dst_v[pl.ds(gi * _L, _L)], 0)
        wvec = jnp.where(valid, w_v[pl.ds(gi * _L, _L)], _NEG)
        for k in range(_L):
            d = dvec[k]
            x = wvec[k]
            cur = m_loc[pl.ds(d, _L)]
            m_loc[pl.ds(d, _L)] = jnp.where(lanes == 0,
                                            jnp.maximum(cur, x), cur)
        return _u
    lax.fori_loop(0, ngrp, grp, 0)

    if True:
        pltpu.sync_copy(m_loc.at[pl.ds(0, N)], sp_ref.at[s])
        plsc.subcore_barrier()
        off = _tile_slices(s)
        pltpu.sync_copy(sp_ref.at[:, pl.ds(off, 640)], mrows)

        def redgrp(j, _u):
            acc = mrows[0, pl.ds(j * _L, _L)]
            for r in range(1, _NS):
                acc = jnp.maximum(acc, mrows[r, pl.ds(j * _L, _L)])
            mout[pl.ds(j * _L, _L)] = acc
            return _u
        lax.fori_loop(0, 640 // _L, redgrp, 0)

        @pl.when(s == _NS - 1)
        def _():
            pltpu.sync_copy(mout, m2_hbm.at[c, pl.ds(off, 640)])

        @pl.when(s < _NS - 1)
        def _():
            pltpu.sync_copy(mout.at[pl.ds(0, 624)],
                            m2_hbm.at[c, pl.ds(off, 624)])


def _segmax_sc(w, dst, n):
    E = w.shape[0]
    epw = E // _NW
    mesh = plsc.VectorSubcoreMesh(core_axis_name="c", subcore_axis_name="s",
                                  num_cores=_NC, num_subcores=_NS)
    f = functools.partial(
        pl.kernel, _segmax_body,
        out_type=jax.ShapeDtypeStruct((_NC, n), jnp.float32),
        mesh=mesh,
        compiler_params=pltpu.CompilerParams(use_tc_tiling_on_sc=False,
                                             needs_layout_passes=False),
        scratch_types=[
            pltpu.VMEM((epw + _L,), jnp.float32),
            pltpu.VMEM((epw + _L,), jnp.int32),
            pltpu.VMEM((n + _L,), jnp.float32),
            pltpu.VMEM((_NS, 640), jnp.float32),
            pltpu.VMEM((640,), jnp.float32),
            pltpu.VMEM_SHARED((_NS, n), jnp.float32),
        ])()
    return f(w, dst)


def _expsum_body(w_hbm, dst_hbm, m2_hbm, ex_hbm, ss2_hbm,
                 w_v, dst_v, ex_v, m_loc, ss_loc, m2v, mrows, mout, sp_ref):
    E = w_hbm.shape[0]
    N = m2_hbm.shape[1]
    epw = E // _NW
    ngrp = (epw + _L - 1) // _L
    c = lax.axis_index("c")
    s = lax.axis_index("s")
    wid = s * _NC + c
    base = wid * epw
    lanes = lax.iota(jnp.int32, _L)

    pltpu.sync_copy(w_hbm.at[pl.ds(base, epw)], w_v.at[pl.ds(0, epw)])
    pltpu.sync_copy(dst_hbm.at[pl.ds(base, epw)], dst_v.at[pl.ds(0, epw)])
    pltpu.sync_copy(m2_hbm, m2v)

    def initloop(i, _u):
        sl = pl.ds(i * _L, _L)
        ss_loc[sl] = jnp.zeros((_L,), jnp.float32)
        return _u
    lax.fori_loop(0, (N + _L) // _L, initloop, 0)

    def mloop(i, _u):
        sl = pl.ds(i * _L, _L)
        m_loc[sl] = jnp.maximum(m2v[0, sl], m2v[1, sl])
        return _u
    lax.fori_loop(0, N // _L, mloop, 0)

    def grp(gi, _u):
        valid = (gi * _L + lanes) < epw
        sl = pl.ds(gi * _L, _L)
        dvec = jnp.where(valid, dst_v[sl], 0)
        wvec = w_v[sl]
        mg = plsc.load_gather(m_loc, [dvec])
        exv = jnp.where(valid, jnp.exp(wvec - mg), 0.0)
        ex_v[sl] = exv
        for k in range(_L):
            d = dvec[k]
            x = exv[k]
            cur = ss_loc[pl.ds(d, _L)]
            ss_loc[pl.ds(d, _L)] = jnp.where(lanes == 0, cur + x, cur)
        return _u
    lax.fori_loop(0, ngrp, grp, 0)

    pltpu.sync_copy(ex_v.at[pl.ds(0, epw)], ex_hbm.at[pl.ds(base, epw)])

    if True:
        pltpu.sync_copy(ss_loc.at[pl.ds(0, N)], sp_ref.at[s])
        plsc.subcore_barrier()
        off = _tile_slices(s)
        pltpu.sync_copy(sp_ref.at[:, pl.ds(off, 640)], mrows)

        def redgrp(j, _u):
            acc = mrows[0, pl.ds(j * _L, _L)]
            for r in range(1, _NS):
                acc = acc + mrows[r, pl.ds(j * _L, _L)]
            mout[pl.ds(j * _L, _L)] = acc
            return _u
        lax.fori_loop(0, 640 // _L, redgrp, 0)

        @pl.when(s == _NS - 1)
        def _():
            pltpu.sync_copy(mout, ss2_hbm.at[c, pl.ds(off, 640)])

        @pl.when(s < _NS - 1)
        def _():
            pltpu.sync_copy(mout.at[pl.ds(0, 624)],
                            ss2_hbm.at[c, pl.ds(off, 624)])


def _expsum_sc(w, dst, m2, n):
    E = w.shape[0]
    epw = E // _NW
    mesh = plsc.VectorSubcoreMesh(core_axis_name="c", subcore_axis_name="s",
                                  num_cores=_NC, num_subcores=_NS)
    f = functools.partial(
        pl.kernel, _expsum_body,
        out_type=[jax.ShapeDtypeStruct((E,), jnp.float32),
                  jax.ShapeDtypeStruct((_NC, n), jnp.float32)],
        mesh=mesh,
        compiler_params=pltpu.CompilerParams(use_tc_tiling_on_sc=False,
                                             needs_layout_passes=False),
        scratch_types=[
            pltpu.VMEM((epw + _L,), jnp.float32),
            pltpu.VMEM((epw + _L,), jnp.int32),
            pltpu.VMEM((epw + _L,), jnp.float32),
            pltpu.VMEM((n + _L,), jnp.float32),
            pltpu.VMEM((n + _L,), jnp.float32),
            pltpu.VMEM((_NC, n), jnp.float32),
            pltpu.VMEM((_NS, 640), jnp.float32),
            pltpu.VMEM((640,), jnp.float32),
            pltpu.VMEM_SHARED((_NS, n), jnp.float32),
        ])()
    return f(w, dst, m2)


_DC = 80


def _message_body(fr_hbm, e3_hbm, gsrc_hbm, dst_hbm, ex_hbm, ss2_hbm,
                  acc_hbm,
                  sidx, didx, gidx, exv, f_rows, e_rows, prod, abuf,
                  ss_loc, ss2buf, zbuf, sem_f, acc_sp):
    E = dst_hbm.shape[0]
    N = ss2_hbm.shape[1]
    H = fr_hbm.shape[1]
    ept = E // _NS
    nchunk = ept // _DC
    c = lax.axis_index("c")
    s = lax.axis_index("s")
    base = s * ept
    lanes = lax.iota(jnp.int32, _L)
    off = _tile_slices(s)

    pltpu.sync_copy(ss2_hbm.at[0], ss_loc.at[pl.ds(0, N)])
    CH = 1000
    for piece in range(N // CH):
        pltpu.sync_copy(ss2_hbm.at[1, pl.ds(piece * CH, CH)], ss2buf)

        def addloop(i, _u, _p=piece):
            sl_d = pl.ds(_p * CH + i * _L, _L)
            sl_s = pl.ds(i * _L, _L)
            ss_loc[sl_d] = ss_loc[sl_d] + ss2buf[sl_s]
            return _u
        lax.fori_loop(0, CH // _L, addloop, 0)

    def zloop(i, _u):
        for j0 in range(H // _L):
            zbuf[i, pl.ds(j0 * _L, _L)] = jnp.zeros((_L,), jnp.float32)
        return _u
    lax.fori_loop(0, _L, zloop, 0)

    def zcopy(tt, _u):
        pltpu.sync_copy(zbuf, acc_sp.at[pl.ds(off + tt * _L, _L), :])
        return _u
    lax.fori_loop(0, 40, zcopy, 0)
    plsc.subcore_barrier()

    def chunk(g, _u):
        b = base + g * _DC
        pltpu.sync_copy(gsrc_hbm.at[pl.ds(c * E + b, _DC)], gidx)
        pltpu.sync_copy(dst_hbm.at[pl.ds(b, _DC)], didx)
        pltpu.sync_copy(ex_hbm.at[pl.ds(b, _DC)], exv)

        cp = pltpu.async_copy(fr_hbm.at[gidx], f_rows, sem_f)
        pltpu.sync_copy(e3_hbm.at[pl.ds(c * E + b, _DC)], e_rows)
        cp.wait()

        def agrp(gi, _v):
            sl = pl.ds(gi * _L, _L)
            ssg = plsc.load_gather(ss_loc, [didx[sl]])
            abuf[sl] = exv[sl] / ssg
            return _v
        lax.fori_loop(0, _DC // _L, agrp, 0)

        def rowgrp(gi, _v):
            av = abuf[pl.ds(gi * _L, _L)]
            for k in range(_L):
                i = gi * _L + k
                a_s = av[k]
                for j0 in range(H // _L):
                    sl = pl.ds(j0 * _L, _L)
                    prod[i, sl] = a_s * (f_rows[i, sl] * e_rows[i, sl])
            return _v
        lax.fori_loop(0, _DC // _L, rowgrp, 0)

        for q in range(_DC // _L):
            dvec = didx[pl.ds(q * _L, _L)]
            pltpu.sync_copy(prod.at[pl.ds(q * _L, _L), :],
                            acc_sp.at[dvec], add=True)
        return _u
    lax.fori_loop(0, nchunk, chunk, 0)

    plsc.subcore_barrier()

    @pl.when(s == _NS - 1)
    def _():
        pltpu.sync_copy(acc_sp.at[pl.ds(off, 640), :],
                        acc_hbm.at[pl.ds(c * N + off, 640)])

    @pl.when(s < _NS - 1)
    def _():
        pltpu.sync_copy(acc_sp.at[pl.ds(off, 624), :],
                        acc_hbm.at[pl.ds(c * N + off, 624)])


def _message_sc(feat, e3, src, dst, ex, ss2):
    N, D = feat.shape
    E = src.shape[0]
    H = D // 2
    fr = feat.reshape(2 * N, H)
    mesh = plsc.VectorSubcoreMesh(core_axis_name="c", subcore_axis_name="s",
                                  num_cores=_NC, num_subcores=_NS)
    f = functools.partial(
        pl.kernel, _message_body,
        out_type=jax.ShapeDtypeStruct((2 * N, H), jnp.float32),
        mesh=mesh,
        compiler_params=pltpu.CompilerParams(use_tc_tiling_on_sc=False,
                                             needs_layout_passes=False),
        scratch_types=[
            pltpu.VMEM((_DC,), jnp.int32),
            pltpu.VMEM((_DC,), jnp.int32),
            pltpu.VMEM((_DC,), jnp.int32),
            pltpu.VMEM((_DC,), jnp.float32),
            pltpu.VMEM((_DC, H), jnp.float32),
            pltpu.VMEM((_DC, H), jnp.float32),
            pltpu.VMEM((_DC, H), jnp.float32),
            pltpu.VMEM((_DC,), jnp.float32),
            pltpu.VMEM((N + _L,), jnp.float32),
            pltpu.VMEM((1000,), jnp.float32),
            pltpu.VMEM((_L, H), jnp.float32),
            pltpu.SemaphoreType.DMA,
            pltpu.VMEM_SHARED((N, H), jnp.float32),
        ])()
    gsrc = jnp.concatenate([2 * src, 2 * src + 1])
    acc2 = f(fr, e3.reshape(2 * E, H), gsrc, dst, ex, ss2)
    return jnp.concatenate([acc2[:N], acc2[N:]], axis=1)


def kernel(feat, edge_index, edge_weight, W_neigh, W_dst, W_self, W_edge, b_edge,
           W_prj_src, b_prj_src, W_prj_dst, b_prj_dst, W_prj_edge, b_prj_edge,
           W_att, b_att, prelu_alpha, out_bias):
    src = edge_index[0]
    dst = edge_index[1]
    n = feat.shape[0]
    d = feat.shape[1]

    Wn = jnp.concatenate([W_prj_src.T, W_prj_dst.T, W_dst.T, W_self.T], axis=1)
    bn = jnp.concatenate([b_prj_src, b_prj_dst,
                          jnp.zeros_like(b_prj_src), out_bias])[None, :]
    hw_src, hw_dst, h_dst, self_out = _mm4(feat, Wn, bn, 2000)

    We = jnp.concatenate([W_prj_edge.T, W_edge.T], axis=1)
    be = jnp.concatenate([b_prj_edge, b_edge])[None, :]
    ew, e = _mm2(edge_weight, We, be, 2000)

    watt = W_att[0]
    alpha16 = jnp.broadcast_to(prelu_alpha, (_L,)).astype(jnp.float32)
    w = _logits_sc(hw_src, hw_dst, ew, src, dst, watt, alpha16)

    m2 = _segmax_sc(w, dst, n)
    ex, ss2 = _expsum_sc(w, dst, m2, n)

    acc = _message_sc(feat, e, src, dst, ex, ss2)

    return _final(self_out, h_dst, acc, W_neigh.T, 2000)


# speedup vs baseline: 3.3312x; 1.0445x over previous
import functools

import jax
import jax.numpy as jnp
from jax import lax
from jax.experimental import pallas as pl
from jax.experimental.pallas import tpu as pltpu
from jax.experimental.pallas import tpu_sc as plsc

_NC = 2
_NS = 16
_NW = _NC * _NS
_L = 16


def _mm_bias_body(x_ref, w_ref, b_ref, o_ref):
    o_ref[...] = jnp.dot(x_ref[...], w_ref[...],
                         preferred_element_type=jnp.float32) + b_ref[...]


def _mm(x, w, b, block_rows):
    M, K = x.shape
    _, Nc = w.shape
    return pl.pallas_call(
        _mm_bias_body,
        grid=(M // block_rows,),
        in_specs=[pl.BlockSpec((block_rows, K), lambda i: (i, 0)),
                  pl.BlockSpec((K, Nc), lambda i: (0, 0)),
                  pl.BlockSpec((1, Nc), lambda i: (0, 0))],
        out_specs=pl.BlockSpec((block_rows, Nc), lambda i: (i, 0)),
        out_shape=jax.ShapeDtypeStruct((M, Nc), jnp.float32),
    )(x, w, b)


def _mm4_body(x_ref, w_ref, b_ref, o1, o2, o3, o4):
    d = o1.shape[1]
    r = jnp.dot(x_ref[...], w_ref[...],
                preferred_element_type=jnp.float32) + b_ref[...]
    o1[...] = r[:, 0 * d:1 * d]
    o2[...] = r[:, 1 * d:2 * d]
    o3[...] = r[:, 2 * d:3 * d]
    o4[...] = r[:, 3 * d:4 * d]


def _mm4(x, w, b, block_rows):
    M, K = x.shape
    _, Nc = w.shape
    d = Nc // 4
    spec = pl.BlockSpec((block_rows, d), lambda i: (i, 0))
    return pl.pallas_call(
        _mm4_body,
        grid=(M // block_rows,),
        in_specs=[pl.BlockSpec((block_rows, K), lambda i: (i, 0)),
                  pl.BlockSpec((K, Nc), lambda i: (0, 0)),
                  pl.BlockSpec((1, Nc), lambda i: (0, 0))],
        out_specs=[spec, spec, spec, spec],
        out_shape=[jax.ShapeDtypeStruct((M, d), jnp.float32)] * 4,
    )(x, w, b)


def _mm2_body(x_ref, w_ref, b_ref, o1, o2):
    d = o1.shape[1]
    h = d // 2
    r = jnp.dot(x_ref[...], w_ref[...],
                preferred_element_type=jnp.float32) + b_ref[...]
    o1[...] = r[:, 0 * d:1 * d]
    o2[0] = r[:, d:d + h]
    o2[1] = r[:, d + h:2 * d]


def _mm2(x, w, b, block_rows):
    M, K = x.shape
    _, Nc = w.shape
    d = Nc // 2
    h = d // 2
    return pl.pallas_call(
        _mm2_body,
        grid=(M // block_rows,),
        in_specs=[pl.BlockSpec((block_rows, K), lambda i: (i, 0)),
                  pl.BlockSpec((K, Nc), lambda i: (0, 0)),
                  pl.BlockSpec((1, Nc), lambda i: (0, 0))],
        out_specs=[pl.BlockSpec((block_rows, d), lambda i: (i, 0)),
                   pl.BlockSpec((2, block_rows, h), lambda i: (0, i, 0))],
        out_shape=[jax.ShapeDtypeStruct((M, d), jnp.float32),
                   jax.ShapeDtypeStruct((2, M, h), jnp.float32)],
    )(x, w, b)


def _final_body(s_ref, hd_ref, acc_ref, w_ref, o_ref):
    prod = hd_ref[...] * acc_ref[...]
    o_ref[...] = s_ref[...] + jnp.dot(prod, w_ref[...],
                                      preferred_element_type=jnp.float32)


def _final(self_out, h_dst, acc, w_neigh_t, block_rows):
    M, D = self_out.shape
    return pl.pallas_call(
        _final_body,
        grid=(M // block_rows,),
        in_specs=[pl.BlockSpec((block_rows, D), lambda i: (i, 0)),
                  pl.BlockSpec((block_rows, D), lambda i: (i, 0)),
                  pl.BlockSpec((block_rows, D), lambda i: (i, 0)),
                  pl.BlockSpec((D, D), lambda i: (0, 0))],
        out_specs=pl.BlockSpec((block_rows, D), lambda i: (i, 0)),
        out_shape=jax.ShapeDtypeStruct((M, D), jnp.float32),
    )(self_out, h_dst, acc, w_neigh_t)


_AC = 128
_ATAIL = 8


def _logits_body(hs_hbm, hd_hbm, ew_hbm, src_hbm, dst_hbm, watt_hbm, alpha_hbm,
                 w_hbm,
                 sidx, didx, s_rows, d_rows, ew_rows, wout,
                 sidx_t, didx_t, s_rows_t, d_rows_t, ew_rows_t, wout_t,
                 watt_v, alpha_v, sem_s, sem_d):
    D = hs_hbm.shape[1]
    E = src_hbm.shape[0]
    epw = E // _NW
    nchunk = (epw - _ATAIL) // _AC
    c = lax.axis_index("c")
    s = lax.axis_index("s")
    wid = s * _NC + c
    base = wid * epw

    pltpu.sync_copy(watt_hbm, watt_v)
    pltpu.sync_copy(alpha_hbm, alpha_v)
    alpha = alpha_v[...]
    lanes = lax.iota(jnp.int32, _L)

    nsub = D // _L
    wsubs = [watt_v[pl.ds(j0 * _L, _L)] for j0 in range(nsub)]

    def compute_group(srows, drows, erows, wo, gi, nvalid):
        def edge(k, wvec):
            i = gi * _L + k
            if nvalid < _L:
                i = jnp.minimum(i, nvalid - 1)
            acc = jnp.zeros((_L,), jnp.float32)
            for j0 in range(nsub):
                sl = pl.ds(j0 * _L, _L)
                z = srows[i, sl] + drows[i, sl] + erows[i, sl]
                p = jnp.maximum(z, 0.0) + alpha * jnp.minimum(z, 0.0)
                acc = acc + wsubs[j0] * p
            tot = jnp.sum(acc)
            return jnp.where(lanes == k, tot, wvec)

        wvec = lax.fori_loop(0, _L, edge, jnp.zeros((_L,), jnp.float32))
        wo[pl.ds(gi * _L, _L)] = wvec

    def chunk(g, _):
        b = base + g * _AC
        pltpu.sync_copy(src_hbm.at[pl.ds(b, _AC)], sidx)
        pltpu.sync_copy(dst_hbm.at[pl.ds(b, _AC)], didx)
        cp1 = pltpu.async_copy(hs_hbm.at[sidx], s_rows, sem_s)
        cp2 = pltpu.async_copy(hd_hbm.at[didx], d_rows, sem_d)
        pltpu.sync_copy(ew_hbm.at[pl.ds(b, _AC)], ew_rows)
        cp1.wait()
        cp2.wait()
        def grp(gi, _u):
            compute_group(s_rows, d_rows, ew_rows, wout, gi, _L)
            return _u
        lax.fori_loop(0, _AC // _L, grp, 0)
        pltpu.sync_copy(wout, w_hbm.at[pl.ds(b, _AC)])
        return _

    lax.fori_loop(0, nchunk, chunk, 0)

    bt = base + nchunk * _AC
    pltpu.sync_copy(src_hbm.at[pl.ds(bt, _ATAIL)], sidx_t)
    pltpu.sync_copy(dst_hbm.at[pl.ds(bt, _ATAIL)], didx_t)
    cp1 = pltpu.async_copy(hs_hbm.at[sidx_t], s_rows_t, sem_s)
    cp2 = pltpu.async_copy(hd_hbm.at[didx_t], d_rows_t, sem_d)
    pltpu.sync_copy(ew_hbm.at[pl.ds(bt, _ATAIL)], ew_rows_t)
    cp1.wait()
    cp2.wait()
    compute_group(s_rows_t, d_rows_t, ew_rows_t, wout_t, 0, _ATAIL)
    pltpu.sync_copy(wout_t.at[pl.ds(0, _ATAIL)], w_hbm.at[pl.ds(bt, _ATAIL)])


def _logits_sc(hs, hd, ew, src, dst, watt, alpha):
    E = src.shape[0]
    D = hs.shape[1]
    mesh = plsc.VectorSubcoreMesh(core_axis_name="c", subcore_axis_name="s",
                                  num_cores=_NC, num_subcores=_NS)
    f = functools.partial(
        pl.kernel, _logits_body,
        out_type=jax.ShapeDtypeStruct((E,), jnp.float32),
        mesh=mesh,
        compiler_params=pltpu.CompilerParams(use_tc_tiling_on_sc=False, needs_layout_passes=False),
        scratch_types=[
            pltpu.VMEM((_AC,), jnp.int32),
            pltpu.VMEM((_AC,), jnp.int32),
            pltpu.VMEM((_AC, D), jnp.float32),
            pltpu.VMEM((_AC, D), jnp.float32),
            pltpu.VMEM((_AC, D), jnp.float32),
            pltpu.VMEM((_AC,), jnp.float32),
            pltpu.VMEM((_ATAIL,), jnp.int32),
            pltpu.VMEM((_ATAIL,), jnp.int32),
            pltpu.VMEM((_ATAIL, D), jnp.float32),
            pltpu.VMEM((_ATAIL, D), jnp.float32),
            pltpu.VMEM((_ATAIL, D), jnp.float32),
            pltpu.VMEM((_L,), jnp.float32),
            pltpu.VMEM((D,), jnp.float32),
            pltpu.VMEM((_L,), jnp.float32),
            pltpu.SemaphoreType.DMA,
            pltpu.SemaphoreType.DMA,
        ])()
    return f(hs, hd, ew, src, dst, watt, alpha)


_NEG = -3.4e38


def _tile_slices(s):
    return s * 624


def _segmax_body(w_hbm, dst_hbm, m2_hbm, w_v, dst_v, m_loc, mrows, mout, sp_ref):
    E = w_hbm.shape[0]
    N = m2_hbm.shape[1]
    epw = E // _NW
    ngrp = (epw + _L - 1) // _L
    c = lax.axis_index("c")
    s = lax.axis_index("s")
    wid = s * _NC + c
    base = wid * epw
    lanes = lax.iota(jnp.int32, _L)

    pltpu.sync_copy(w_hbm.at[pl.ds(base, epw)], w_v.at[pl.ds(0, epw)])
    pltpu.sync_copy(dst_hbm.at[pl.ds(base, epw)], dst_v.at[pl.ds(0, epw)])

    def initloop(i, _u):
        m_loc[pl.ds(i * _L, _L)] = jnp.full((_L,), _NEG, jnp.float32)
        return _u
    lax.fori_loop(0, (N + _L) // _L, initloop, 0)

    def grp(gi, _u):
        valid = (gi * _L + lanes) < epw
        dvec = jnp.where(valid, dst_v[pl.ds(gi * _L, _L)], 0)
        wvec = jnp.where(valid, w_v[pl.ds(gi * _L, _L)], _NEG)
        for k in range(_L):
            d = dvec[k]
            x = wvec[k]
            cur = m_loc[pl.ds(d, _L)]
            m_loc[pl.ds(d, _L)] = jnp.where(lanes == 0,
                                            jnp.maximum(cur, x), cur)
        return _u
    lax.fori_loop(0, ngrp, grp, 0)

    if True:
        pltpu.sync_copy(m_loc.at[pl.ds(0, N)], sp_ref.at[s])
        plsc.subcore_barrier()
        off = _tile_slices(s)
        pltpu.sync_copy(sp_ref.at[:, pl.ds(off, 640)], mrows)

        def redgrp(j, _u):
            acc = mrows[0, pl.ds(j * _L, _L)]
            for r in range(1, _NS):
                acc = jnp.maximum(acc, mrows[r, pl.ds(j * _L, _L)])
            mout[pl.ds(j * _L, _L)] = acc
            return _u
        lax.fori_loop(0, 640 // _L, redgrp, 0)

        @pl.when(s == _NS - 1)
        def _():
            pltpu.sync_copy(mout, m2_hbm.at[c, pl.ds(off, 640)])

        @pl.when(s < _NS - 1)
        def _():
            pltpu.sync_copy(mout.at[pl.ds(0, 624)],
                            m2_hbm.at[c, pl.ds(off, 624)])


def _segmax_sc(w, dst, n):
    E = w.shape[0]
    epw = E // _NW
    mesh = plsc.VectorSubcoreMesh(core_axis_name="c", subcore_axis_name="s",
                                  num_cores=_NC, num_subcores=_NS)
    f = functools.partial(
        pl.kernel, _segmax_body,
        out_type=jax.ShapeDtypeStruct((_NC, n), jnp.float32),
        mesh=mesh,
        compiler_params=pltpu.CompilerParams(use_tc_tiling_on_sc=False,
                                             needs_layout_passes=False),
        scratch_types=[
            pltpu.VMEM((epw + _L,), jnp.float32),
            pltpu.VMEM((epw + _L,), jnp.int32),
            pltpu.VMEM((n + _L,), jnp.float32),
            pltpu.VMEM((_NS, 640), jnp.float32),
            pltpu.VMEM((640,), jnp.float32),
            pltpu.VMEM_SHARED((_NS, n), jnp.float32),
        ])()
    return f(w, dst)


def _expsum_body(w_hbm, dst_hbm, m2_hbm, ex_hbm, ss2_hbm,
                 w_v, dst_v, ex_v, m_loc, ss_loc, m2v, mrows, mout, sp_ref):
    E = w_hbm.shape[0]
    N = m2_hbm.shape[1]
    epw = E // _NW
    ngrp = (epw + _L - 1) // _L
    c = lax.axis_index("c")
    s = lax.axis_index("s")
    wid = s * _NC + c
    base = wid * epw
    lanes = lax.iota(jnp.int32, _L)

    pltpu.sync_copy(w_hbm.at[pl.ds(base, epw)], w_v.at[pl.ds(0, epw)])
    pltpu.sync_copy(dst_hbm.at[pl.ds(base, epw)], dst_v.at[pl.ds(0, epw)])
    pltpu.sync_copy(m2_hbm, m2v)

    def initloop(i, _u):
        sl = pl.ds(i * _L, _L)
        ss_loc[sl] = jnp.zeros((_L,), jnp.float32)
        return _u
    lax.fori_loop(0, (N + _L) // _L, initloop, 0)

    def mloop(i, _u):
        sl = pl.ds(i * _L, _L)
        m_loc[sl] = jnp.maximum(m2v[0, sl], m2v[1, sl])
        return _u
    lax.fori_loop(0, N // _L, mloop, 0)

    def grp(gi, _u):
        valid = (gi * _L + lanes) < epw
        sl = pl.ds(gi * _L, _L)
        dvec = jnp.where(valid, dst_v[sl], 0)
        wvec = w_v[sl]
        mg = plsc.load_gather(m_loc, [dvec])
        exv = jnp.where(valid, jnp.exp(wvec - mg), 0.0)
        ex_v[sl] = exv
        for k in range(_L):
            d = dvec[k]
            x = exv[k]
            cur = ss_loc[pl.ds(d, _L)]
            ss_loc[pl.ds(d, _L)] = jnp.where(lanes == 0, cur + x, cur)
        return _u
    lax.fori_loop(0, ngrp, grp, 0)

    pltpu.sync_copy(ex_v.at[pl.ds(0, epw)], ex_hbm.at[pl.ds(base, epw)])

    if True:
        pltpu.sync_copy(ss_loc.at[pl.ds(0, N)], sp_ref.at[s])
        plsc.subcore_barrier()
        off = _tile_slices(s)
        pltpu.sync_copy(sp_ref.at[:, pl.ds(off, 640)], mrows)

        def redgrp(j, _u):
            acc = mrows[0, pl.ds(j * _L, _L)]
            for r in range(1, _NS):
                acc = acc + mrows[r, pl.ds(j * _L, _L)]
            mout[pl.ds(j * _L, _L)] = acc
            return _u
        lax.fori_loop(0, 640 // _L, redgrp, 0)

        @pl.when(s == _NS - 1)
        def _():
            pltpu.sync_copy(mout, ss2_hbm.at[c, pl.ds(off, 640)])

        @pl.when(s < _NS - 1)
        def _():
            pltpu.sync_copy(mout.at[pl.ds(0, 624)],
                            ss2_hbm.at[c, pl.ds(off, 624)])


def _expsum_sc(w, dst, m2, n):
    E = w.shape[0]
    epw = E // _NW
    mesh = plsc.VectorSubcoreMesh(core_axis_name="c", subcore_axis_name="s",
                                  num_cores=_NC, num_subcores=_NS)
    f = functools.partial(
        pl.kernel, _expsum_body,
        out_type=[jax.ShapeDtypeStruct((E,), jnp.float32),
                  jax.ShapeDtypeStruct((_NC, n), jnp.float32)],
        mesh=mesh,
        compiler_params=pltpu.CompilerParams(use_tc_tiling_on_sc=False,
                                             needs_layout_passes=False),
        scratch_types=[
            pltpu.VMEM((epw + _L,), jnp.float32),
            pltpu.VMEM((epw + _L,), jnp.int32),
            pltpu.VMEM((epw + _L,), jnp.float32),
            pltpu.VMEM((n + _L,), jnp.float32),
            pltpu.VMEM((n + _L,), jnp.float32),
            pltpu.VMEM((_NC, n), jnp.float32),
            pltpu.VMEM((_NS, 640), jnp.float32),
            pltpu.VMEM((640,), jnp.float32),
            pltpu.VMEM_SHARED((_NS, n), jnp.float32),
        ])()
    return f(w, dst, m2)


_DC = 80


def _message_body(fr_hbm, e3_hbm, gsrc_hbm, dst_hbm, ex_hbm, ssum_hbm,
                  acc_hbm,
                  sidx, didx, gidx, exv, f_rows, e_rows, prod, abuf,
                  ss_loc, zbuf, sem_f, acc_sp):
    E = dst_hbm.shape[0]
    N = ssum_hbm.shape[0]
    H = fr_hbm.shape[1]
    ept = E // _NS
    nchunk = ept // _DC
    c = lax.axis_index("c")
    s = lax.axis_index("s")
    base = s * ept
    lanes = lax.iota(jnp.int32, _L)
    off = _tile_slices(s)

    pltpu.sync_copy(ssum_hbm, ss_loc.at[pl.ds(0, N)])

    def zloop(i, _u):
        for j0 in range(H // _L):
            zbuf[i, pl.ds(j0 * _L, _L)] = jnp.zeros((_L,), jnp.float32)
        return _u
    lax.fori_loop(0, _L, zloop, 0)

    def zcopy(tt, _u):
        pltpu.sync_copy(zbuf, acc_sp.at[pl.ds(off + tt * _L, _L), :])
        return _u
    lax.fori_loop(0, 40, zcopy, 0)
    plsc.subcore_barrier()

    def chunk(g, _u):
        b = base + g * _DC
        pltpu.sync_copy(gsrc_hbm.at[pl.ds(c * E + b, _DC)], gidx)
        pltpu.sync_copy(dst_hbm.at[pl.ds(b, _DC)], didx)
        pltpu.sync_copy(ex_hbm.at[pl.ds(b, _DC)], exv)

        cp = pltpu.async_copy(fr_hbm.at[gidx], f_rows, sem_f)
        pltpu.sync_copy(e3_hbm.at[pl.ds(c * E + b, _DC)], e_rows)
        cp.wait()

        def agrp(gi, _v):
            sl = pl.ds(gi * _L, _L)
            ssg = plsc.load_gather(ss_loc, [didx[sl]])
            abuf[sl] = exv[sl] / ssg
            return _v
        lax.fori_loop(0, _DC // _L, agrp, 0)

        def rowgrp(gi, _v):
            av = abuf[pl.ds(gi * _L, _L)]
            for k in range(_L):
                i = gi * _L + k
                a_s = av[k]
                for j0 in range(H // _L):
                    sl = pl.ds(j0 * _L, _L)
                    prod[i, sl] = a_s * (f_rows[i, sl] * e_rows[i, sl])
            return _v
        lax.fori_loop(0, _DC // _L, rowgrp, 0)

        for q in range(_DC // _L):
            dvec = didx[pl.ds(q * _L, _L)]
            pltpu.sync_copy(prod.at[pl.ds(q * _L, _L), :],
                            acc_sp.at[dvec], add=True)
        return _u
    lax.fori_loop(0, nchunk, chunk, 0)

    plsc.subcore_barrier()

    @pl.when(s == _NS - 1)
    def _():
        pltpu.sync_copy(acc_sp.at[pl.ds(off, 640), :],
                        acc_hbm.at[pl.ds(c * N + off, 640)])

    @pl.when(s < _NS - 1)
    def _():
        pltpu.sync_copy(acc_sp.at[pl.ds(off, 624), :],
                        acc_hbm.at[pl.ds(c * N + off, 624)])


def _message_sc(feat, e3, src, dst, ex, ssum):
    N, D = feat.shape
    E = src.shape[0]
    H = D // 2
    fr = feat.reshape(2 * N, H)
    mesh = plsc.VectorSubcoreMesh(core_axis_name="c", subcore_axis_name="s",
                                  num_cores=_NC, num_subcores=_NS)
    f = functools.partial(
        pl.kernel, _message_body,
        out_type=jax.ShapeDtypeStruct((2 * N, H), jnp.float32),
        mesh=mesh,
        compiler_params=pltpu.CompilerParams(use_tc_tiling_on_sc=False,
                                             needs_layout_passes=False),
        scratch_types=[
            pltpu.VMEM((_DC,), jnp.int32),
            pltpu.VMEM((_DC,), jnp.int32),
            pltpu.VMEM((_DC,), jnp.int32),
            pltpu.VMEM((_DC,), jnp.float32),
            pltpu.VMEM((_DC, H), jnp.float32),
            pltpu.VMEM((_DC, H), jnp.float32),
            pltpu.VMEM((_DC, H), jnp.float32),
            pltpu.VMEM((_DC,), jnp.float32),
            pltpu.VMEM((N + _L,), jnp.float32),
            pltpu.VMEM((_L, H), jnp.float32),
            pltpu.SemaphoreType.DMA,
            pltpu.VMEM_SHARED((N, H), jnp.float32),
        ])()
    gsrc = jnp.concatenate([2 * src, 2 * src + 1])
    acc2 = f(fr, e3.reshape(2 * E, H), gsrc, dst, ex, ssum)
    return jnp.concatenate([acc2[:N], acc2[N:]], axis=1)


def kernel(feat, edge_index, edge_weight, W_neigh, W_dst, W_self, W_edge, b_edge,
           W_prj_src, b_prj_src, W_prj_dst, b_prj_dst, W_prj_edge, b_prj_edge,
           W_att, b_att, prelu_alpha, out_bias):
    src = edge_index[0]
    dst = edge_index[1]
    n = feat.shape[0]
    d = feat.shape[1]

    Wn = jnp.concatenate([W_prj_src.T, W_prj_dst.T, W_dst.T, W_self.T], axis=1)
    bn = jnp.concatenate([b_prj_src, b_prj_dst,
                          jnp.zeros_like(b_prj_src), out_bias])[None, :]
    hw_src, hw_dst, h_dst, self_out = _mm4(feat, Wn, bn, 2000)

    We = jnp.concatenate([W_prj_edge.T, W_edge.T], axis=1)
    be = jnp.concatenate([b_prj_edge, b_edge])[None, :]
    ew, e = _mm2(edge_weight, We, be, 2000)

    watt = W_att[0]
    alpha16 = jnp.broadcast_to(prelu_alpha, (_L,)).astype(jnp.float32)
    w = _logits_sc(hw_src, hw_dst, ew, src, dst, watt, alpha16)

    m2 = _segmax_sc(w, dst, n)
    ex, ss2 = _expsum_sc(w, dst, m2, n)

    acc = _message_sc(feat, e, src, dst, ex, ss2[0] + ss2[1])

    return _final(self_out, h_dst, acc, W_neigh.T, 2000)
